# Initial kernel scaffold; baseline (speedup 1.0000x reference)
import functools

import jax
import jax.numpy as jnp
from jax import lax
from jax.experimental import pallas as pl
from jax.experimental.pallas import tpu as pltpu
from jax.experimental.pallas import tpu_sc as plsc

N = 10000
D = 128
E = 320000

NC = 2
NS = 16
NW = NC * NS

CHUNK = 128
CPT = 80
EP = NW * CPT * CHUNK
NPAD = 10240
RPS = NPAD // NS
NB = 4

_mesh = plsc.VectorSubcoreMesh(
    core_axis_name="c", subcore_axis_name="s", num_cores=NC, num_subcores=NS
)


def _sc_deg_body(dst_hbm, z1_hbm, degp_hbm, dstv, ones_v, dacc, sem):
    del sem
    c = lax.axis_index("c")
    s = lax.axis_index("s")
    wid = c * NS + s
    for i in range(CHUNK // 16):
        ones_v[pl.ds(i * 16, 16)] = jnp.ones((16,), jnp.float32)
    pltpu.sync_copy(z1_hbm.at[pl.ds(s * RPS, RPS)], dacc.at[pl.ds(s * RPS, RPS)])
    plsc.subcore_barrier()
    pltpu.sync_copy(dst_hbm.at[wid], dstv)

    @pl.loop(0, CPT)
    def _(j):
        pltpu.sync_copy(ones_v, dacc.at[dstv.at[j]], add=True)

    plsc.subcore_barrier()
    pltpu.sync_copy(dacc.at[pl.ds(s * RPS, RPS)], degp_hbm.at[c, pl.ds(s * RPS, RPS)])


@jax.jit
def _sc_deg(dst3, zeros1):
    kern = pl.kernel(
        _sc_deg_body,
        out_type=jax.ShapeDtypeStruct((NC, NPAD), jnp.float32),
        mesh=_mesh,
        scratch_types=[
            pltpu.VMEM((CPT, CHUNK), jnp.int32),
            pltpu.VMEM((CHUNK,), jnp.float32),
            pltpu.VMEM_SHARED((NPAD,), jnp.float32),
            pltpu.SemaphoreType.DMA,
        ],
    )
    return kern(dst3, zeros1)


def _sc_agg_body(hs_hbm, src_hbm, dst_hbm, z2_hbm, out_hbm,
                 srcv, dstv, b0, b1, b2, b3, acc, s0, s1, s2, s3):
    c = lax.axis_index("c")
    s = lax.axis_index("s")
    wid = c * NS + s
    bufs = [b0, b1, b2, b3]
    sems = [s0, s1, s2, s3]
    pltpu.sync_copy(z2_hbm.at[pl.ds(s * RPS, RPS)], acc.at[pl.ds(s * RPS, RPS)])
    plsc.subcore_barrier()
    pltpu.sync_copy(src_hbm.at[wid], srcv)
    pltpu.sync_copy(dst_hbm.at[wid], dstv)

    for b in range(NB):
        pltpu.async_copy(hs_hbm.at[srcv.at[b]], bufs[b], sems[b])

    @pl.loop(0, CPT, step=NB)
    def _(j0):
        for b in range(NB):
            jj = j0 + b
            pltpu.make_async_copy(hs_hbm.at[srcv.at[jj]], bufs[b], sems[b]).wait()
            pltpu.sync_copy(bufs[b], acc.at[dstv.at[jj]], add=True)
            nxt = jj + NB

            @pl.when(nxt < CPT)
            def _():
                pltpu.async_copy(hs_hbm.at[srcv.at[nxt]], bufs[b], sems[b])

    plsc.subcore_barrier()
    pltpu.sync_copy(acc.at[pl.ds(s * RPS, RPS)], out_hbm.at[c].at[pl.ds(s * RPS, RPS)])


@jax.jit
def _sc_agg(hs, src3, dst3, zeros2):
    kern = pl.kernel(
        _sc_agg_body,
        out_type=jax.ShapeDtypeStruct((NC, NPAD, D), jnp.float32),
        mesh=_mesh,
        scratch_types=[
            pltpu.VMEM((CPT, CHUNK), jnp.int32),
            pltpu.VMEM((CPT, CHUNK), jnp.int32),
            pltpu.VMEM((CHUNK, D), jnp.float32),
            pltpu.VMEM((CHUNK, D), jnp.float32),
            pltpu.VMEM((CHUNK, D), jnp.float32),
            pltpu.VMEM((CHUNK, D), jnp.float32),
            pltpu.VMEM_SHARED((NPAD, D), jnp.float32),
            pltpu.SemaphoreType.DMA,
            pltpu.SemaphoreType.DMA,
            pltpu.SemaphoreType.DMA,
            pltpu.SemaphoreType.DMA,
        ],
    )
    return kern(hs, src3, dst3, zeros2)


def _tc1_body(degp_ref, x_ref, w_ref, dinv_ref, hs_ref):
    deg = degp_ref[0, :N] + degp_ref[1, :N] + 1.0
    di = lax.rsqrt(deg)[:, None]
    dinv_ref[...] = di
    hs_ref[...] = (
        jnp.dot(x_ref[...], w_ref[...], preferred_element_type=jnp.float32) * di
    )


def _tc2_body(accp_ref, hs_ref, dinv_ref, b_ref, w_ref, out_ref):
    di = dinv_ref[...]
    z = (accp_ref[0, :N, :] + accp_ref[1, :N, :] + hs_ref[...]) * di + b_ref[...]
    h = jnp.maximum(z, 0.0)
    out_ref[...] = (
        jnp.dot(h, w_ref[...], preferred_element_type=jnp.float32) * di
    )


def _tc3_body(accp_ref, hs_ref, dinv_ref, b_ref, out_ref):
    di = dinv_ref[...]
    out_ref[...] = (
        (accp_ref[0, :N, :] + accp_ref[1, :N, :] + hs_ref[...]) * di + b_ref[...]
    )


@jax.jit
def _run(x, src3, dst3, W1, b1, W2, b2, zeros1, zeros2):
    degp = _sc_deg(dst3, zeros1)
    dinv, hs1 = pl.pallas_call(
        _tc1_body,
        out_shape=(
            jax.ShapeDtypeStruct((N, 1), jnp.float32),
            jax.ShapeDtypeStruct((N, D), jnp.float32),
        ),
    )(degp, x, W1)
    acc1 = _sc_agg(hs1, src3, dst3, zeros2)
    hs2 = pl.pallas_call(
        _tc2_body,
        out_shape=jax.ShapeDtypeStruct((N, D), jnp.float32),
    )(acc1, hs1, dinv, b1, W2)
    acc2 = _sc_agg(hs2, src3, dst3, zeros2)
    out = pl.pallas_call(
        _tc3_body,
        out_shape=jax.ShapeDtypeStruct((N, D), jnp.float32),
    )(acc2, hs2, dinv, b2)
    return out


def kernel(x, edge_index, W1, b1, W2, b2):
    src = edge_index[0].astype(jnp.int32)
    dst = edge_index[1].astype(jnp.int32)
    pad = EP - E
    src3 = jnp.concatenate([src, jnp.zeros((pad,), jnp.int32)]).reshape(NW, CPT, CHUNK)
    dst3 = jnp.concatenate([dst, jnp.full((pad,), N, jnp.int32)]).reshape(NW, CPT, CHUNK)
    zeros1 = jnp.zeros((NPAD,), jnp.float32)
    zeros2 = jnp.zeros((NPAD, D), jnp.float32)
    return _run(x, src3, dst3, W1, b1, W2, b2, zeros1, zeros2)


# speedup vs baseline: 14.2741x; 14.2741x over previous
import jax
import jax.numpy as jnp
from jax import lax
from jax.experimental import pallas as pl
from jax.experimental.pallas import tpu as pltpu
from jax.experimental.pallas import tpu_sc as plsc

N = 10000
D = 128
E = 320000

NC = 2
NS = 16
HD = D // NC

CHUNK = 128
CPE = 160
HCPE = CPE // NC
EP = NS * CPE * CHUNK
NPAD = 10240
RPS = NPAD // NS
NB = 4

_mesh = plsc.VectorSubcoreMesh(
    core_axis_name="c", subcore_axis_name="s", num_cores=NC, num_subcores=NS
)


def _sc_deg_body(dst_hbm, z1_hbm, degp_hbm, dstv, ones_v, dacc, sem):
    del sem
    c = lax.axis_index("c")
    s = lax.axis_index("s")
    for i in range(CHUNK // 16):
        ones_v[pl.ds(i * 16, 16)] = jnp.ones((16,), jnp.float32)
    pltpu.sync_copy(z1_hbm.at[pl.ds(s * RPS, RPS)], dacc.at[pl.ds(s * RPS, RPS)])
    plsc.subcore_barrier()
    pltpu.sync_copy(dst_hbm.at[s].at[pl.ds(c * HCPE, HCPE)], dstv)

    @pl.loop(0, HCPE)
    def _(j):
        pltpu.sync_copy(ones_v, dacc.at[dstv.at[j]], add=True)

    plsc.subcore_barrier()
    pltpu.sync_copy(dacc.at[pl.ds(s * RPS, RPS)], degp_hbm.at[c, pl.ds(s * RPS, RPS)])


@jax.jit
def _sc_deg(dst3, zeros1):
    kern = pl.kernel(
        _sc_deg_body,
        out_type=jax.ShapeDtypeStruct((NC, NPAD), jnp.float32),
        mesh=_mesh,
        scratch_types=[
            pltpu.VMEM((HCPE, CHUNK), jnp.int32),
            pltpu.VMEM((CHUNK,), jnp.float32),
            pltpu.VMEM_SHARED((NPAD,), jnp.float32),
            pltpu.SemaphoreType.DMA,
        ],
    )
    return kern(dst3, zeros1)


def _sc_agg_body(hs_hbm, src_hbm, dst_hbm, z2_hbm, out_hbm,
                 srcv, dstv, b0, b1, b2, b3, acc, s0, s1, s2, s3):
    c = lax.axis_index("c")
    s = lax.axis_index("s")
    bufs = [b0, b1, b2, b3]
    sems = [s0, s1, s2, s3]
    tbl = hs_hbm.at[c]
    pltpu.sync_copy(z2_hbm.at[pl.ds(s * RPS, RPS)], acc.at[pl.ds(s * RPS, RPS)])
    plsc.subcore_barrier()
    pltpu.sync_copy(src_hbm.at[s], srcv)
    pltpu.sync_copy(dst_hbm.at[s], dstv)

    for b in range(NB):
        pltpu.async_copy(tbl.at[srcv.at[b]], bufs[b], sems[b])

    @pl.loop(0, CPE, step=NB)
    def _(j0):
        for b in range(NB):
            jj = j0 + b
            pltpu.make_async_copy(tbl.at[srcv.at[jj]], bufs[b], sems[b]).wait()
            pltpu.sync_copy(bufs[b], acc.at[dstv.at[jj]], add=True)
            nxt = jj + NB

            @pl.when(nxt < CPE)
            def _():
                pltpu.async_copy(tbl.at[srcv.at[nxt]], bufs[b], sems[b])

    plsc.subcore_barrier()
    pltpu.sync_copy(acc.at[pl.ds(s * RPS, RPS)], out_hbm.at[c].at[pl.ds(s * RPS, RPS)])


@jax.jit
def _sc_agg(hs_split, src3, dst3, zeros2):
    kern = pl.kernel(
        _sc_agg_body,
        out_type=jax.ShapeDtypeStruct((NC, NPAD, HD), jnp.float32),
        mesh=_mesh,
        compiler_params=pltpu.CompilerParams(use_tc_tiling_on_sc=False),
        scratch_types=[
            pltpu.VMEM((CPE, CHUNK), jnp.int32),
            pltpu.VMEM((CPE, CHUNK), jnp.int32),
            pltpu.VMEM((CHUNK, HD), jnp.float32),
            pltpu.VMEM((CHUNK, HD), jnp.float32),
            pltpu.VMEM((CHUNK, HD), jnp.float32),
            pltpu.VMEM((CHUNK, HD), jnp.float32),
            pltpu.VMEM_SHARED((NPAD, HD), jnp.float32),
            pltpu.SemaphoreType.DMA,
            pltpu.SemaphoreType.DMA,
            pltpu.SemaphoreType.DMA,
            pltpu.SemaphoreType.DMA,
        ],
    )
    return kern(hs_split, src3, dst3, zeros2)


def _tc1_body(degp_ref, x_ref, w_ref, dinv_ref, hs_ref):
    deg = degp_ref[0, :N] + degp_ref[1, :N] + 1.0
    di = lax.rsqrt(deg)[:, None]
    dinv_ref[...] = di
    h = jnp.dot(x_ref[...], w_ref[...], preferred_element_type=jnp.float32) * di
    hs_ref[0, :, :] = h[:, :HD]
    hs_ref[1, :, :] = h[:, HD:]


def _tc2_body(accp_ref, hs_ref, dinv_ref, b_ref, w_ref, out_ref):
    di = dinv_ref[...]
    a = jnp.concatenate(
        [accp_ref[0, :N, :] + hs_ref[0, :, :], accp_ref[1, :N, :] + hs_ref[1, :, :]],
        axis=-1,
    )
    z = a * di + b_ref[...]
    h = jnp.maximum(z, 0.0)
    hs2 = jnp.dot(h, w_ref[...], preferred_element_type=jnp.float32) * di
    out_ref[0, :, :] = hs2[:, :HD]
    out_ref[1, :, :] = hs2[:, HD:]


def _tc3_body(accp_ref, hs_ref, dinv_ref, b_ref, out_ref):
    di = dinv_ref[...]
    a = jnp.concatenate(
        [accp_ref[0, :N, :] + hs_ref[0, :, :], accp_ref[1, :N, :] + hs_ref[1, :, :]],
        axis=-1,
    )
    out_ref[...] = a * di + b_ref[...]


@jax.jit
def _run(x, src3, dst3, W1, b1, W2, b2, zeros1, zeros2):
    degp = _sc_deg(dst3, zeros1)
    dinv, hs1 = pl.pallas_call(
        _tc1_body,
        out_shape=(
            jax.ShapeDtypeStruct((N, 1), jnp.float32),
            jax.ShapeDtypeStruct((NC, N, HD), jnp.float32),
        ),
    )(degp, x, W1)
    acc1 = _sc_agg(hs1, src3, dst3, zeros2)
    hs2 = pl.pallas_call(
        _tc2_body,
        out_shape=jax.ShapeDtypeStruct((NC, N, HD), jnp.float32),
    )(acc1, hs1, dinv, b1, W2)
    acc2 = _sc_agg(hs2, src3, dst3, zeros2)
    out = pl.pallas_call(
        _tc3_body,
        out_shape=jax.ShapeDtypeStruct((N, D), jnp.float32),
    )(acc2, hs2, dinv, b2)
    return out


def kernel(x, edge_index, W1, b1, W2, b2):
    src = edge_index[0].astype(jnp.int32)
    dst = edge_index[1].astype(jnp.int32)
    pad = EP - E
    src3 = jnp.concatenate([src, jnp.zeros((pad,), jnp.int32)]).reshape(NS, CPE, CHUNK)
    dst3 = jnp.concatenate([dst, jnp.full((pad,), N, jnp.int32)]).reshape(NS, CPE, CHUNK)
    zeros1 = jnp.zeros((NPAD,), jnp.float32)
    zeros2 = jnp.zeros((NPAD, HD), jnp.float32)
    return _run(x, src3, dst3, W1, b1, W2, b2, zeros1, zeros2)
